# baseline (speedup 1.0000x reference)
import jax
import jax.numpy as jnp
from jax import lax
from jax.experimental import pallas as pl
from jax.experimental.pallas import tpu as pltpu
from jax.experimental.pallas import tpu_sc as plsc

_N = 50000
_E = 800000
_F = 8
_H = 16
_GROW = 128
_UNIT = 2048
_GPS = _UNIT // _GROW
_EROWS = _E // _GROW
_PROWS = 6400
_EP = _PROWS * _GROW
_NUNITS = _EP // _UNIT
_NC = 2
_NS = 16
_NW = _NC * _NS
_KMAX = -(-_NUNITS // _NW)
_TROWS = 3128
_TROWS_LAST = _N - 15 * _TROWS

_B2 = 400
_EBLK = _B2 * _GROW
_NEB = _EP // _EBLK
_B2C = 200
_EBLKC = _B2C * _GROW
_NEBC = _EP // _EBLKC
_NBLK = 5000
_NNB = _N // _NBLK

_mesh = plsc.VectorSubcoreMesh(core_axis_name="c", subcore_axis_name="s",
                               num_cores=_NC, num_subcores=_NS)
_sc_params = pltpu.CompilerParams(use_tc_tiling_on_sc=False,
                                  needs_layout_passes=False)


def _transpose_rows(rows, trT):
    lane = lax.iota(jnp.int32, 16)

    def grp(r, c):
        for gg in range(8):
            ridx = (r * 8 + gg) * 16 + lane
            for f in range(_F):
                x = plsc.load_gather(
                    rows, [ridx, jnp.full((16,), f, jnp.int32)])
                trT[f, r, pl.ds(gg * 16, 16)] = x
        return c

    lax.fori_loop(0, _GPS, grp, 0)


def _sc_gather_body(v_hbm, src_hbm, dst_hbm, vs_out, vd_out,
                    idx_s, idx_d, rows_s, rows_d, trT_s, trT_d, sem):
    cid = lax.axis_index("c")
    sid = lax.axis_index("s")
    wid = sid * _NC + cid

    def unit_body(k, carry):
        u = wid + k * _NW

        @pl.when(u < _NUNITS)
        def _():
            erow = u * _GPS
            pltpu.sync_copy(src_hbm.at[pl.ds(erow, _GPS)], idx_s)
            pltpu.sync_copy(dst_hbm.at[pl.ds(erow, _GPS)], idx_d)
            for g in range(2):
                descs = []
                for j in range(8):
                    jj = g * 8 + j
                    descs.append(pltpu.async_copy(
                        v_hbm.at[idx_s.at[jj]],
                        rows_s.at[pl.ds(jj * _GROW, _GROW)], sem))
                    descs.append(pltpu.async_copy(
                        v_hbm.at[idx_d.at[jj]],
                        rows_d.at[pl.ds(jj * _GROW, _GROW)], sem))
                for dsc in descs:
                    dsc.wait()

            _transpose_rows(rows_s, trT_s)
            _transpose_rows(rows_d, trT_d)
            pltpu.sync_copy(trT_s, vs_out.at[:, pl.ds(erow, _GPS), :])
            pltpu.sync_copy(trT_d, vd_out.at[:, pl.ds(erow, _GPS), :])

        return carry

    lax.fori_loop(0, _KMAX, unit_body, 0)


def _gather_call(v, src2d, dst2d):
    return pl.kernel(
        _sc_gather_body,
        out_type=[jax.ShapeDtypeStruct((_F, _PROWS, _GROW), jnp.float32),
                  jax.ShapeDtypeStruct((_F, _PROWS, _GROW), jnp.float32)],
        mesh=_mesh,
        scratch_types=[pltpu.VMEM((_GPS, _GROW), jnp.int32),
                       pltpu.VMEM((_GPS, _GROW), jnp.int32),
                       pltpu.VMEM((_UNIT, _F), jnp.float32),
                       pltpu.VMEM((_UNIT, _F), jnp.float32),
                       pltpu.VMEM((_F, _GPS, _GROW), jnp.float32),
                       pltpu.VMEM((_F, _GPS, _GROW), jnp.float32),
                       pltpu.SemaphoreType.DMA],
        compiler_params=_sc_params,
    )(v, src2d, dst2d)


def _sc_scatter_body(msg_hbm, dst_hbm, z8_hbm, out_hbm, cnt_out,
                     idx_v, rows_v, mT_v, sem, acc, cnt):
    cid = lax.axis_index("c")
    sid = lax.axis_index("s")
    wid = sid * _NC + cid
    lane = lax.iota(jnp.int32, 16)
    ones16 = jnp.ones((16,), jnp.float32)

    @pl.when(sid < _NS - 1)
    def _():
        pltpu.sync_copy(z8_hbm, acc.at[pl.ds(sid * _TROWS, _TROWS)])

    @pl.when(sid == _NS - 1)
    def _():
        pltpu.sync_copy(z8_hbm.at[pl.ds(0, _TROWS_LAST)],
                        acc.at[pl.ds((_NS - 1) * _TROWS, _TROWS_LAST)])

    def zcnt(i, c):
        cnt[pl.ds(i * 16, 16)] = jnp.zeros((16,), jnp.float32)
        return c

    lax.fori_loop(0, _N // 16, zcnt, 0)
    plsc.subcore_barrier()

    def unit_body(k, carry):
        u = wid + k * _NW

        @pl.when(u * _GPS < _EROWS)
        def _():
            pltpu.sync_copy(dst_hbm.at[pl.ds(u * _GPS, _GPS)], idx_v)
            pltpu.sync_copy(msg_hbm.at[:, pl.ds(u * _GPS, _GPS), :], mT_v)

            @pl.when(u == _EROWS // _GPS)
            def _():
                for f in range(_F):
                    for r in range(10, _GPS):
                        for cc in range(8):
                            mT_v[f, r, pl.ds(cc * 16, 16)] = (
                                jnp.zeros((16,), jnp.float32))

            def grp(r, c2):
                rfull = jnp.full((16,), 0, jnp.int32) + r
                for gg in range(8):
                    eidx = (r * 8 + gg) * 16 + lane
                    for f in range(_F):
                        x = plsc.load_gather(
                            mT_v, [jnp.full((16,), f, jnp.int32), rfull,
                                   gg * 16 + lane])
                        plsc.store_scatter(
                            rows_v, [eidx, jnp.full((16,), f, jnp.int32)], x)
                return c2

            lax.fori_loop(0, _GPS, grp, 0)

            descs = []
            for j in range(_GPS):
                descs.append(pltpu.async_copy(
                    rows_v.at[pl.ds(j * _GROW, _GROW)],
                    acc.at[idx_v.at[j]], sem, add=True))
            for j in range(_GPS):
                @pl.when(u * _GPS + j < _EROWS)
                def _():
                    for h in range(8):
                        ii = idx_v[j, pl.ds(h * 16, 16)]
                        plsc.addupdate_scatter(cnt, [ii], ones16)
            for dsc in descs:
                dsc.wait()

        return carry

    lax.fori_loop(0, _KMAX, unit_body, 0)
    pltpu.sync_copy(cnt, cnt_out.at[wid])
    plsc.subcore_barrier()

    @pl.when(sid < _NS - 1)
    def _():
        pltpu.sync_copy(acc.at[pl.ds(sid * _TROWS, _TROWS)],
                        out_hbm.at[cid, pl.ds(sid * _TROWS, _TROWS)])

    @pl.when(sid == _NS - 1)
    def _():
        pltpu.sync_copy(acc.at[pl.ds((_NS - 1) * _TROWS, _TROWS_LAST)],
                        out_hbm.at[cid, pl.ds((_NS - 1) * _TROWS, _TROWS_LAST)])


def _scatter_call(msgT3, dst2d, z8):
    return pl.kernel(
        _sc_scatter_body,
        out_type=[jax.ShapeDtypeStruct((_NC, _N, _F), jnp.float32),
                  jax.ShapeDtypeStruct((_NW, _N), jnp.float32)],
        mesh=_mesh,
        scratch_types=[pltpu.VMEM((_GPS, _GROW), jnp.int32),
                       pltpu.VMEM((_UNIT, _F), jnp.float32),
                       pltpu.VMEM((_F, _GPS, _GROW), jnp.float32),
                       pltpu.SemaphoreType.DMA,
                       pltpu.VMEM_SHARED((_N, _F), jnp.float32),
                       pltpu.VMEM((_N,), jnp.float32)],
        compiler_params=_sc_params,
    )(msgT3, dst2d, z8)


def _pass_a_body(vs3, vd3, ea, w1, b1, h1o, s1o, s2o):
    i = pl.program_id(0)
    vsT = vs3[...].reshape(_F, _EBLK)
    vdT = vd3[...].reshape(_F, _EBLK)
    eaT = ea[...]
    dT = vdT - vsT
    u2 = jnp.sum(dT * dT, axis=0, keepdims=True)
    u = jnp.sqrt(u2)
    vdirT = dT / jnp.maximum(u, 1e-12)
    eT = jnp.concatenate(
        [eaT, u, vdirT, jnp.zeros((5, _EBLK), jnp.float32)], axis=0)
    h1 = lax.dot_general(w1[...], eT, (((1,), (0,)), ((), ())),
                         preferred_element_type=jnp.float32) + b1[:, 0:1]
    h1o[...] = h1
    pos = i * _EBLK + lax.broadcasted_iota(jnp.int32, (1, _EBLK), 1)
    hm = jnp.where(pos < _E, h1, 0.0)

    @pl.when(i == 0)
    def _():
        s1o[...] = jnp.zeros_like(s1o)
        s2o[...] = jnp.zeros_like(s2o)

    s1o[...] = s1o[...] + jnp.sum(hm, axis=1, keepdims=True)
    s2o[...] = s2o[...] + jnp.sum(hm * h1, axis=1, keepdims=True)


def _pass_a_call(vsT3, vdT3, eaT, W1tp, b1p):
    return pl.pallas_call(
        _pass_a_body,
        grid=(_NEB,),
        in_specs=[pl.BlockSpec((_F, _B2, _GROW), lambda i: (0, i, 0)),
                  pl.BlockSpec((_F, _B2, _GROW), lambda i: (0, i, 0)),
                  pl.BlockSpec((2, _EBLK), lambda i: (0, i)),
                  pl.BlockSpec((_H, _H), lambda i: (0, 0)),
                  pl.BlockSpec((_H, 128), lambda i: (0, 0))],
        out_specs=[pl.BlockSpec((_H, _EBLK), lambda i: (0, i)),
                   pl.BlockSpec((_H, 128), lambda i: (0, 0)),
                   pl.BlockSpec((_H, 128), lambda i: (0, 0))],
        out_shape=[jax.ShapeDtypeStruct((_H, _EP), jnp.float32),
                   jax.ShapeDtypeStruct((_H, 128), jnp.float32),
                   jax.ShapeDtypeStruct((_H, 128), jnp.float32)],
    )(vsT3, vdT3, eaT, W1tp, b1p)


def _pass_b_body(h1, sc1, sh1, llo, slo):
    i = pl.program_id(0)
    L = h1[...] * sc1[:, 0:1] + sh1[:, 0:1]
    L = jnp.where(L >= 0, L, 0.01 * L)
    pos = i * _EBLK + lax.broadcasted_iota(jnp.int32, (1, _EBLK), 1)
    L = jnp.where(pos < _E, L, 0.0)
    ll = lax.dot_general(L, L, (((1,), (1,)), ((), ())),
                         preferred_element_type=jnp.float32)

    @pl.when(i == 0)
    def _():
        llo[...] = jnp.zeros_like(llo)
        slo[...] = jnp.zeros_like(slo)

    llo[...] = llo[...] + ll
    slo[...] = slo[...] + jnp.sum(L, axis=1, keepdims=True)


def _pass_b_call(h1T, sc1p, sh1p):
    return pl.pallas_call(
        _pass_b_body,
        grid=(_NEB,),
        in_specs=[pl.BlockSpec((_H, _EBLK), lambda i: (0, i)),
                  pl.BlockSpec((_H, 128), lambda i: (0, 0)),
                  pl.BlockSpec((_H, 128), lambda i: (0, 0))],
        out_specs=[pl.BlockSpec((_H, _H), lambda i: (0, 0)),
                   pl.BlockSpec((_H, 128), lambda i: (0, 0))],
        out_shape=[jax.ShapeDtypeStruct((_H, _H), jnp.float32),
                   jax.ShapeDtypeStruct((_H, 128), jnp.float32)],
    )(h1T, sc1p, sh1p)


def _pass_c_body(h1, vs3, sc1, sh1, w2t, c2, mo):
    L = h1[...] * sc1[:, 0:1] + sh1[:, 0:1]
    L = jnp.where(L >= 0, L, 0.01 * L)
    h2 = lax.dot_general(w2t[...], L, (((1,), (0,)), ((), ())),
                         preferred_element_type=jnp.float32) + c2[:, 0:1]
    th = jnp.tanh(h2)
    vsT = vs3[...].reshape(_F, _EBLKC)
    m = vsT[0:1, :] * th[0:_F, :]
    for j in range(1, _F):
        m = m + vsT[j:j + 1, :] * th[_F * j:_F * (j + 1), :]
    mo[...] = m.reshape(_F, _B2C, _GROW)


def _pass_c_call(h1T, vsT3, sc1p, sh1p, W2t, c2p):
    return pl.pallas_call(
        _pass_c_body,
        grid=(_NEBC,),
        in_specs=[pl.BlockSpec((_H, _EBLKC), lambda i: (0, i)),
                  pl.BlockSpec((_F, _B2C, _GROW), lambda i: (0, i, 0)),
                  pl.BlockSpec((_H, 128), lambda i: (0, 0)),
                  pl.BlockSpec((_H, 128), lambda i: (0, 0)),
                  pl.BlockSpec((64, _H), lambda i: (0, 0)),
                  pl.BlockSpec((64, 128), lambda i: (0, 0))],
        out_specs=pl.BlockSpec((_F, _B2C, _GROW), lambda i: (0, i, 0)),
        out_shape=jax.ShapeDtypeStruct((_F, _PROWS, _GROW), jnp.float32),
    )(h1T, vsT3, sc1p, sh1p, W2t, c2p)


def _pass_d_body(pa, pb, cb, vb, rt, bi, oo):
    s = pa[...] + pb[...]
    cnt = jnp.maximum(cb[...], 1.0)
    agg = s / cnt
    x = vb[...]
    o = agg + jnp.dot(x, rt[...], preferred_element_type=jnp.float32) + bi[0:1, :]
    o = jnp.where(o >= 0, o, 0.01 * o)
    oo[...] = o + x


def _pass_d_call(pa, pb, cnts, v, rootp, biasp):
    return pl.pallas_call(
        _pass_d_body,
        grid=(_NNB,),
        in_specs=[pl.BlockSpec((_NBLK, _F), lambda i: (i, 0)),
                  pl.BlockSpec((_NBLK, _F), lambda i: (i, 0)),
                  pl.BlockSpec((_NBLK, 1), lambda i: (i, 0)),
                  pl.BlockSpec((_NBLK, _F), lambda i: (i, 0)),
                  pl.BlockSpec((_F, _F), lambda i: (0, 0)),
                  pl.BlockSpec((8, _F), lambda i: (0, 0))],
        out_specs=pl.BlockSpec((_NBLK, _F), lambda i: (i, 0)),
        out_shape=jax.ShapeDtypeStruct((_N, _F), jnp.float32),
    )(pa, pb, cnts, v, rootp, biasp)


def kernel(v, edge_index, edge_attr, W1, b1, g1, be1, W2, b2, g2, be2,
           root, bias):
    f32 = jnp.float32
    pad = jnp.zeros((_EP - _E,), jnp.int32)
    src2d = jnp.concatenate([edge_index[0], pad]).reshape(_PROWS, _GROW)
    dst2d = jnp.concatenate([edge_index[1], pad]).reshape(_PROWS, _GROW)

    vsT3, vdT3 = _gather_call(v, src2d, dst2d)
    eaT = jnp.zeros((2, _EP), f32).at[:, 0:_E].set(edge_attr.T)

    W1tp = jnp.zeros((_H, _H), f32).at[:, 0:11].set(W1.T)
    b1p = jnp.broadcast_to(b1[:, None], (_H, 128))
    h1T, s1o, s2o = _pass_a_call(vsT3, vdT3, eaT, W1tp, b1p)

    m1 = s1o[:, 0] / _E
    v1 = s2o[:, 0] / _E - m1 * m1
    inv1 = lax.rsqrt(v1 + 1e-5)
    sc1 = g1 * inv1
    sh1 = be1 - m1 * sc1
    sc1p = jnp.broadcast_to(sc1[:, None], (_H, 128))
    sh1p = jnp.broadcast_to(sh1[:, None], (_H, 128))

    ll, slo = _pass_b_call(h1T, sc1p, sh1p)

    sL = slo[:, 0]
    mL = sL / _E
    mW = mL @ W2
    m2 = mW + b2
    t = (ll / _E) @ W2
    v2 = jnp.sum(W2 * t, axis=0) - mW * mW
    inv2 = lax.rsqrt(v2 + 1e-5)
    sc2 = g2 * inv2
    W2t = (W2 * sc2[None, :]).T
    c2 = (b2 - m2) * sc2 + be2
    c2p = jnp.broadcast_to(c2[:, None], (64, 128))

    msgT3 = _pass_c_call(h1T, vsT3, sc1p, sh1p, W2t, c2p)

    z8 = jnp.zeros((_TROWS, _F), f32)
    parts, cnts = _scatter_call(msgT3, dst2d, z8)

    cnt1 = jnp.sum(cnts, axis=0)[:, None]
    biasp = jnp.broadcast_to(bias[None, :], (8, _F))
    return _pass_d_call(parts[0], parts[1], cnt1, v, root, biasp)


# speedup vs baseline: 1.0404x; 1.0404x over previous
import jax
import jax.numpy as jnp
from jax import lax
from jax.experimental import pallas as pl
from jax.experimental.pallas import tpu as pltpu
from jax.experimental.pallas import tpu_sc as plsc

_N = 50000
_E = 800000
_F = 8
_H = 16
_GROW = 128
_UNIT = 2048
_GPS = _UNIT // _GROW
_EROWS = _E // _GROW
_PROWS = 6400
_EP = _PROWS * _GROW
_NUNITS = _EP // _UNIT
_NC = 2
_NS = 16
_NW = _NC * _NS
_KMAX = -(-_NUNITS // _NW)
_TROWS = 3128
_TROWS_LAST = _N - 15 * _TROWS

_B2 = 400
_EBLK = _B2 * _GROW
_NEB = _EP // _EBLK
_B2C = 200
_EBLKC = _B2C * _GROW
_NEBC = _EP // _EBLKC
_NBLK = 5000
_NNB = _N // _NBLK

_mesh = plsc.VectorSubcoreMesh(core_axis_name="c", subcore_axis_name="s",
                               num_cores=_NC, num_subcores=_NS)
_sc_params = pltpu.CompilerParams(use_tc_tiling_on_sc=False,
                                  needs_layout_passes=False)


def _transpose_rows(rows, trT):
    lane = lax.iota(jnp.int32, 16)

    def grp(r, c):
        for gg in range(8):
            ridx = (r * 8 + gg) * 16 + lane
            for f in range(_F):
                x = plsc.load_gather(
                    rows, [ridx, jnp.full((16,), f, jnp.int32)])
                trT[f, r, pl.ds(gg * 16, 16)] = x
        return c

    lax.fori_loop(0, _GPS, grp, 0)


def _sc_gather_body(v_hbm, src_hbm, dst_hbm, vs_out, vd_out,
                    idx_s, idx_d, rows_s, rows_d, trT_s, trT_d, sem):
    cid = lax.axis_index("c")
    sid = lax.axis_index("s")
    wid = sid * _NC + cid

    def unit_body(k, carry):
        u = wid + k * _NW

        @pl.when(u < _NUNITS)
        def _():
            erow = u * _GPS
            pltpu.sync_copy(src_hbm.at[pl.ds(erow, _GPS)], idx_s)
            pltpu.sync_copy(dst_hbm.at[pl.ds(erow, _GPS)], idx_d)
            for g in range(2):
                descs = []
                for j in range(8):
                    jj = g * 8 + j
                    descs.append(pltpu.async_copy(
                        v_hbm.at[idx_s.at[jj]],
                        rows_s.at[pl.ds(jj * _GROW, _GROW)], sem))
                    descs.append(pltpu.async_copy(
                        v_hbm.at[idx_d.at[jj]],
                        rows_d.at[pl.ds(jj * _GROW, _GROW)], sem))
                for dsc in descs:
                    dsc.wait()

            _transpose_rows(rows_s, trT_s)
            _transpose_rows(rows_d, trT_d)
            pltpu.sync_copy(trT_s, vs_out.at[:, pl.ds(erow, _GPS), :])
            pltpu.sync_copy(trT_d, vd_out.at[:, pl.ds(erow, _GPS), :])

        return carry

    lax.fori_loop(0, _KMAX, unit_body, 0)


def _gather_call(v, src2d, dst2d):
    return pl.kernel(
        _sc_gather_body,
        out_type=[jax.ShapeDtypeStruct((_F, _PROWS, _GROW), jnp.float32),
                  jax.ShapeDtypeStruct((_F, _PROWS, _GROW), jnp.float32)],
        mesh=_mesh,
        scratch_types=[pltpu.VMEM((_GPS, _GROW), jnp.int32),
                       pltpu.VMEM((_GPS, _GROW), jnp.int32),
                       pltpu.VMEM((_UNIT, _F), jnp.float32),
                       pltpu.VMEM((_UNIT, _F), jnp.float32),
                       pltpu.VMEM((_F, _GPS, _GROW), jnp.float32),
                       pltpu.VMEM((_F, _GPS, _GROW), jnp.float32),
                       pltpu.SemaphoreType.DMA],
        compiler_params=_sc_params,
    )(v, src2d, dst2d)


def _sc_scatter_body(msg_hbm, dst_hbm, out_hbm, idx_v, rows_v, mT_v, sem, acc):
    cid = lax.axis_index("c")
    sid = lax.axis_index("s")
    wid = sid * _NC + cid
    lane = lax.iota(jnp.int32, 16)

    def z(i, c):
        rows_v[i, :] = jnp.zeros((16,), jnp.float32)
        return c

    lax.fori_loop(0, _UNIT, z, 0)
    pltpu.sync_copy(rows_v, acc.at[pl.ds(sid * _TROWS, _UNIT)])

    @pl.when(sid < _NS - 1)
    def _():
        pltpu.sync_copy(rows_v.at[pl.ds(0, _TROWS - _UNIT)],
                        acc.at[pl.ds(sid * _TROWS + _UNIT, _TROWS - _UNIT)])

    @pl.when(sid == _NS - 1)
    def _():
        pltpu.sync_copy(rows_v.at[pl.ds(0, _TROWS_LAST - _UNIT)],
                        acc.at[pl.ds(sid * _TROWS + _UNIT, _TROWS_LAST - _UNIT)])

    cnt_row = jnp.where(lane == _F, 1.0, 0.0).astype(jnp.float32)

    def zc(i, c):
        rows_v[i, :] = cnt_row
        return c

    lax.fori_loop(0, _UNIT, zc, 0)
    plsc.subcore_barrier()

    def unit_body(k, carry):
        u = wid + k * _NW

        @pl.when(u * _GPS < _EROWS)
        def _():
            pltpu.sync_copy(dst_hbm.at[pl.ds(u * _GPS, _GPS)], idx_v)
            pltpu.sync_copy(msg_hbm.at[:, pl.ds(u * _GPS, _GPS), :], mT_v)

            def grp(r, c2):
                rfull = jnp.full((16,), 0, jnp.int32) + r
                for gg in range(8):
                    eidx = (r * 8 + gg) * 16 + lane
                    for f in range(_F):
                        x = plsc.load_gather(
                            mT_v, [jnp.full((16,), f, jnp.int32), rfull,
                                   gg * 16 + lane])
                        plsc.store_scatter(
                            rows_v, [eidx, jnp.full((16,), f, jnp.int32)], x)
                return c2

            lax.fori_loop(0, _GPS, grp, 0)

            @pl.when(u == _EROWS // _GPS)
            def _():
                def zp(i, c):
                    rows_v[i, :] = jnp.zeros((16,), jnp.float32)
                    return c

                lax.fori_loop(_E - (_EROWS // _GPS) * _UNIT, _UNIT, zp, 0)

            descs = []
            for j in range(_GPS):
                descs.append(pltpu.async_copy(
                    rows_v.at[pl.ds(j * _GROW, _GROW)],
                    acc.at[idx_v.at[j]], sem, add=True))
            for dsc in descs:
                dsc.wait()

        return carry

    lax.fori_loop(0, _KMAX, unit_body, 0)
    plsc.subcore_barrier()

    @pl.when(sid < _NS - 1)
    def _():
        pltpu.sync_copy(acc.at[pl.ds(sid * _TROWS, _TROWS)],
                        out_hbm.at[cid, pl.ds(sid * _TROWS, _TROWS)])

    @pl.when(sid == _NS - 1)
    def _():
        pltpu.sync_copy(acc.at[pl.ds((_NS - 1) * _TROWS, _TROWS_LAST)],
                        out_hbm.at[cid, pl.ds((_NS - 1) * _TROWS, _TROWS_LAST)])


def _scatter_call(msgT3, dst2d):
    return pl.kernel(
        _sc_scatter_body,
        out_type=jax.ShapeDtypeStruct((_NC, _N, 16), jnp.float32),
        mesh=_mesh,
        scratch_types=[pltpu.VMEM((_GPS, _GROW), jnp.int32),
                       pltpu.VMEM((_UNIT, 16), jnp.float32),
                       pltpu.VMEM((_F, _GPS, _GROW), jnp.float32),
                       pltpu.SemaphoreType.DMA,
                       pltpu.VMEM_SHARED((_N, 16), jnp.float32)],
        compiler_params=_sc_params,
    )(msgT3, dst2d)


def _pass_a_body(vs3, vd3, ea, w1, b1, h1o, s1o, s2o):
    i = pl.program_id(0)
    vsT = vs3[...].reshape(_F, _EBLK)
    vdT = vd3[...].reshape(_F, _EBLK)
    eaT = ea[...]
    dT = vdT - vsT
    u2 = jnp.sum(dT * dT, axis=0, keepdims=True)
    u = jnp.sqrt(u2)
    vdirT = dT / jnp.maximum(u, 1e-12)
    eT = jnp.concatenate(
        [eaT, u, vdirT, jnp.zeros((5, _EBLK), jnp.float32)], axis=0)
    h1 = lax.dot_general(w1[...], eT, (((1,), (0,)), ((), ())),
                         preferred_element_type=jnp.float32) + b1[:, 0:1]
    h1o[...] = h1
    pos = i * _EBLK + lax.broadcasted_iota(jnp.int32, (1, _EBLK), 1)
    hm = jnp.where(pos < _E, h1, 0.0)

    @pl.when(i == 0)
    def _():
        s1o[...] = jnp.zeros_like(s1o)
        s2o[...] = jnp.zeros_like(s2o)

    s1o[...] = s1o[...] + jnp.sum(hm, axis=1, keepdims=True)
    s2o[...] = s2o[...] + jnp.sum(hm * h1, axis=1, keepdims=True)


def _pass_a_call(vsT3, vdT3, eaT, W1tp, b1p):
    return pl.pallas_call(
        _pass_a_body,
        grid=(_NEB,),
        in_specs=[pl.BlockSpec((_F, _B2, _GROW), lambda i: (0, i, 0)),
                  pl.BlockSpec((_F, _B2, _GROW), lambda i: (0, i, 0)),
                  pl.BlockSpec((2, _EBLK), lambda i: (0, i)),
                  pl.BlockSpec((_H, _H), lambda i: (0, 0)),
                  pl.BlockSpec((_H, 128), lambda i: (0, 0))],
        out_specs=[pl.BlockSpec((_H, _EBLK), lambda i: (0, i)),
                   pl.BlockSpec((_H, 128), lambda i: (0, 0)),
                   pl.BlockSpec((_H, 128), lambda i: (0, 0))],
        out_shape=[jax.ShapeDtypeStruct((_H, _EP), jnp.float32),
                   jax.ShapeDtypeStruct((_H, 128), jnp.float32),
                   jax.ShapeDtypeStruct((_H, 128), jnp.float32)],
    )(vsT3, vdT3, eaT, W1tp, b1p)


def _pass_b_body(h1, sc1, sh1, llo, slo):
    i = pl.program_id(0)
    L = h1[...] * sc1[:, 0:1] + sh1[:, 0:1]
    L = jnp.where(L >= 0, L, 0.01 * L)
    pos = i * _EBLK + lax.broadcasted_iota(jnp.int32, (1, _EBLK), 1)
    L = jnp.where(pos < _E, L, 0.0)
    ll = lax.dot_general(L, L, (((1,), (1,)), ((), ())),
                         preferred_element_type=jnp.float32)

    @pl.when(i == 0)
    def _():
        llo[...] = jnp.zeros_like(llo)
        slo[...] = jnp.zeros_like(slo)

    llo[...] = llo[...] + ll
    slo[...] = slo[...] + jnp.sum(L, axis=1, keepdims=True)


def _pass_b_call(h1T, sc1p, sh1p):
    return pl.pallas_call(
        _pass_b_body,
        grid=(_NEB,),
        in_specs=[pl.BlockSpec((_H, _EBLK), lambda i: (0, i)),
                  pl.BlockSpec((_H, 128), lambda i: (0, 0)),
                  pl.BlockSpec((_H, 128), lambda i: (0, 0))],
        out_specs=[pl.BlockSpec((_H, _H), lambda i: (0, 0)),
                   pl.BlockSpec((_H, 128), lambda i: (0, 0))],
        out_shape=[jax.ShapeDtypeStruct((_H, _H), jnp.float32),
                   jax.ShapeDtypeStruct((_H, 128), jnp.float32)],
    )(h1T, sc1p, sh1p)


def _pass_c_body(h1, vs3, sc1, sh1, w2t, c2, mo):
    L = h1[...] * sc1[:, 0:1] + sh1[:, 0:1]
    L = jnp.where(L >= 0, L, 0.01 * L)
    h2 = lax.dot_general(w2t[...], L, (((1,), (0,)), ((), ())),
                         preferred_element_type=jnp.float32) + c2[:, 0:1]
    th = jnp.tanh(h2)
    vsT = vs3[...].reshape(_F, _EBLKC)
    m = vsT[0:1, :] * th[0:_F, :]
    for j in range(1, _F):
        m = m + vsT[j:j + 1, :] * th[_F * j:_F * (j + 1), :]
    mo[...] = m.reshape(_F, _B2C, _GROW)


def _pass_c_call(h1T, vsT3, sc1p, sh1p, W2t, c2p):
    return pl.pallas_call(
        _pass_c_body,
        grid=(_NEBC,),
        in_specs=[pl.BlockSpec((_H, _EBLKC), lambda i: (0, i)),
                  pl.BlockSpec((_F, _B2C, _GROW), lambda i: (0, i, 0)),
                  pl.BlockSpec((_H, 128), lambda i: (0, 0)),
                  pl.BlockSpec((_H, 128), lambda i: (0, 0)),
                  pl.BlockSpec((64, _H), lambda i: (0, 0)),
                  pl.BlockSpec((64, 128), lambda i: (0, 0))],
        out_specs=pl.BlockSpec((_F, _B2C, _GROW), lambda i: (0, i, 0)),
        out_shape=jax.ShapeDtypeStruct((_F, _PROWS, _GROW), jnp.float32),
    )(h1T, vsT3, sc1p, sh1p, W2t, c2p)


def _pass_d_body(pa, pb, vb, rt, bi, oo):
    s = pa[...] + pb[...]
    cnt = jnp.maximum(s[:, _F:_F + 1], 1.0)
    agg = s[:, 0:_F] / cnt
    x = vb[...]
    o = agg + jnp.dot(x, rt[...], preferred_element_type=jnp.float32) + bi[0:1, :]
    o = jnp.where(o >= 0, o, 0.01 * o)
    oo[...] = o + x


def _pass_d_call(pa, pb, v, rootp, biasp):
    return pl.pallas_call(
        _pass_d_body,
        grid=(_NNB,),
        in_specs=[pl.BlockSpec((_NBLK, 16), lambda i: (i, 0)),
                  pl.BlockSpec((_NBLK, 16), lambda i: (i, 0)),
                  pl.BlockSpec((_NBLK, _F), lambda i: (i, 0)),
                  pl.BlockSpec((_F, _F), lambda i: (0, 0)),
                  pl.BlockSpec((8, _F), lambda i: (0, 0))],
        out_specs=pl.BlockSpec((_NBLK, _F), lambda i: (i, 0)),
        out_shape=jax.ShapeDtypeStruct((_N, _F), jnp.float32),
    )(pa, pb, v, rootp, biasp)


def kernel(v, edge_index, edge_attr, W1, b1, g1, be1, W2, b2, g2, be2,
           root, bias):
    f32 = jnp.float32
    pad = jnp.zeros((_EP - _E,), jnp.int32)
    src2d = jnp.concatenate([edge_index[0], pad]).reshape(_PROWS, _GROW)
    dst2d = jnp.concatenate([edge_index[1], pad]).reshape(_PROWS, _GROW)

    vsT3, vdT3 = _gather_call(v, src2d, dst2d)
    eaT = jnp.zeros((2, _EP), f32).at[:, 0:_E].set(edge_attr.T)

    W1tp = jnp.zeros((_H, _H), f32).at[:, 0:11].set(W1.T)
    b1p = jnp.broadcast_to(b1[:, None], (_H, 128))
    h1T, s1o, s2o = _pass_a_call(vsT3, vdT3, eaT, W1tp, b1p)

    m1 = s1o[:, 0] / _E
    v1 = s2o[:, 0] / _E - m1 * m1
    inv1 = lax.rsqrt(v1 + 1e-5)
    sc1 = g1 * inv1
    sh1 = be1 - m1 * sc1
    sc1p = jnp.broadcast_to(sc1[:, None], (_H, 128))
    sh1p = jnp.broadcast_to(sh1[:, None], (_H, 128))

    ll, slo = _pass_b_call(h1T, sc1p, sh1p)

    sL = slo[:, 0]
    mL = sL / _E
    mW = mL @ W2
    m2 = mW + b2
    t = (ll / _E) @ W2
    v2 = jnp.sum(W2 * t, axis=0) - mW * mW
    inv2 = lax.rsqrt(v2 + 1e-5)
    sc2 = g2 * inv2
    W2t = (W2 * sc2[None, :]).T
    c2 = (b2 - m2) * sc2 + be2
    c2p = jnp.broadcast_to(c2[:, None], (64, 128))

    msgT3 = _pass_c_call(h1T, vsT3, sc1p, sh1p, W2t, c2p)

    parts = _scatter_call(msgT3, dst2d)

    biasp = jnp.broadcast_to(bias[None, :], (8, _F))
    return _pass_d_call(parts[0], parts[1], v, root, biasp)
